# Initial kernel scaffold; baseline (speedup 1.0000x reference)
import jax
import jax.numpy as jnp
from jax.experimental import pallas as pl


def kernel(idx, special_table, attached_table, index_map):
    raise NotImplementedError("write your pallas kernel here")


# speedup vs baseline: 22.7925x; 22.7925x over previous
import functools

import jax
import jax.numpy as jnp
from jax import lax
from jax.experimental import pallas as pl
from jax.experimental.pallas import tpu as pltpu
from jax.experimental.pallas import tpu_sc as plsc

_D = 64
_C = 1024
_K = 128
_NK = _C // _K
_L = 16


@functools.partial(jax.jit, static_argnums=(0, 1))
def _lookup_call(n, num_rows, i0, i1, map_flat, table):
    info = plsc.get_sparse_core_info()
    nw = info.num_cores * info.num_subcores
    assert n % (nw * _C) == 0, (n, nw, _C)
    b_per_w = n // nw
    n_chunks = b_per_w // _C
    mesh = plsc.VectorSubcoreMesh(core_axis_name="c", subcore_axis_name="s")

    @functools.partial(
        pl.kernel,
        mesh=mesh,
        compiler_params=pltpu.CompilerParams(use_tc_tiling_on_sc=False),
        out_type=jax.ShapeDtypeStruct((n, _D), jnp.float32),
        scratch_types=[
            pltpu.VMEM((_C,), jnp.int32),
            pltpu.VMEM((_C,), jnp.int32),
            pltpu.VMEM((_NK, _K), jnp.int32),
            pltpu.VMEM((_NK, _K), jnp.int32),
            pltpu.VMEM((_NK, _K), jnp.int32),
            pltpu.VMEM((_C, _D), jnp.float32),
            pltpu.SemaphoreType.DMA,
        ],
    )
    def lookup(i0_hbm, i1_hbm, map_hbm, table_hbm, out_hbm,
               i0_v, i1_v, lin_v, map_v, row_v, rows_v, sem):
        wid = lax.axis_index("s") * info.num_cores + lax.axis_index("c")
        base = wid * b_per_w

        def chunk_body(g, carry):
            off = base + g * _C
            pltpu.sync_copy(i0_hbm.at[pl.ds(off, _C)], i0_v)
            pltpu.sync_copy(i1_hbm.at[pl.ds(off, _C)], i1_v)
            for j in range(_NK):
                for t in range(_K // _L):
                    sl = pl.ds(j * _K + t * _L, _L)
                    v0 = i0_v[sl]
                    v1 = i1_v[sl]
                    lin_v[j, pl.ds(t * _L, _L)] = v0 * 8 + v1
            hs = [pltpu.async_copy(map_hbm.at[lin_v.at[j]], map_v.at[j], sem)
                  for j in range(_NK)]
            for h in hs:
                h.wait()
            for j in range(_NK):
                for t in range(_K // _L):
                    sl = pl.ds(j * _K + t * _L, _L)
                    v0 = i0_v[sl]
                    m = map_v[j, pl.ds(t * _L, _L)]
                    m = jnp.minimum(jnp.maximum(m, 0), num_rows - 4)
                    spec = (num_rows - 3) + jnp.minimum(jnp.maximum(v0, 0), 2)
                    row_v[j, pl.ds(t * _L, _L)] = jnp.where(v0 <= 2, spec, m)
            hs = [pltpu.async_copy(table_hbm.at[row_v.at[j]],
                                   rows_v.at[pl.ds(j * _K, _K)], sem)
                  for j in range(_NK)]
            for h in hs:
                h.wait()
            pltpu.sync_copy(rows_v, out_hbm.at[pl.ds(off, _C)])
            return carry

        lax.fori_loop(0, n_chunks, chunk_body, 0)

    return lookup(i0, i1, map_flat, table)


def kernel(idx, special_table, attached_table, index_map):
    original_shape = idx.shape[:-1]
    n = 1
    for s in original_shape:
        n *= s
    flat = idx.reshape(n, 2)
    i0 = flat[:, 0].astype(jnp.int32)
    i1 = flat[:, 1].astype(jnp.int32)
    a = attached_table.shape[0]
    table = jnp.concatenate(
        [attached_table.astype(jnp.float32), special_table.astype(jnp.float32)],
        axis=0)
    map_flat = index_map.reshape(-1).astype(jnp.int32)
    out = _lookup_call(n, a + 3, i0, i1, map_flat, table)
    return out.reshape(*original_shape, _D)


# speedup vs baseline: 24.4917x; 1.0746x over previous
import functools

import jax
import jax.numpy as jnp
from jax import lax
from jax.experimental import pallas as pl
from jax.experimental.pallas import tpu as pltpu
from jax.experimental.pallas import tpu_sc as plsc

_D = 64
_C = 512
_K = 128
_NK = _C // _K
_L = 16


@functools.partial(jax.jit, static_argnums=(0, 1, 2))
def _lookup_call(n, num_rows, map_words, i0, i1, map_packed, table):
    info = plsc.get_sparse_core_info()
    nw = info.num_cores * info.num_subcores
    assert n % (nw * 2 * _C) == 0, (n, nw, _C)
    b_per_w = n // nw
    n_pairs = b_per_w // (2 * _C)
    mesh = plsc.VectorSubcoreMesh(core_axis_name="c", subcore_axis_name="s")

    @functools.partial(
        pl.kernel,
        mesh=mesh,
        compiler_params=pltpu.CompilerParams(use_tc_tiling_on_sc=False,
                                             needs_layout_passes=False),
        out_type=jax.ShapeDtypeStruct((n, _D), jnp.float32),
        scratch_types=[
            pltpu.VMEM((map_words,), jnp.int32),
            pltpu.VMEM((2, _C), jnp.int32),
            pltpu.VMEM((2, _C), jnp.int32),
            pltpu.VMEM((2, _NK, _K), jnp.int32),
            pltpu.VMEM((2, _C, _D), jnp.float32),
            pltpu.SemaphoreType.DMA,
            pltpu.SemaphoreType.DMA,
            pltpu.SemaphoreType.DMA,
            pltpu.SemaphoreType.DMA,
        ],
    )
    def lookup(i0_hbm, i1_hbm, map_hbm, table_hbm, out_hbm,
               map_v, i0_v, i1_v, row_v, rows_v, sem_idx, sem_g, sem_o0,
               sem_o1):
        wid = lax.axis_index("s") * info.num_cores + lax.axis_index("c")
        base = wid * b_per_w
        n_chunks = 2 * n_pairs
        sem_o = (sem_o0, sem_o1)

        pltpu.async_copy(i0_hbm.at[pl.ds(base, _C)], i0_v.at[0], sem_idx)
        pltpu.async_copy(i1_hbm.at[pl.ds(base, _C)], i1_v.at[0], sem_idx)
        pltpu.sync_copy(map_hbm, map_v)

        def process(i, g, b):
            off = base + g * _C
            pltpu.make_async_copy(
                i0_hbm.at[pl.ds(0, _C)], i0_v.at[b], sem_idx).wait()
            pltpu.make_async_copy(
                i1_hbm.at[pl.ds(0, _C)], i1_v.at[b], sem_idx).wait()
            off_n = base + jnp.minimum(g + 1, n_chunks - 1) * _C
            pltpu.async_copy(i0_hbm.at[pl.ds(off_n, _C)], i0_v.at[1 - b],
                             sem_idx)
            pltpu.async_copy(i1_hbm.at[pl.ds(off_n, _C)], i1_v.at[1 - b],
                             sem_idx)
            for j in range(_NK):
                for t in range(_K // _L):
                    sl = pl.ds(j * _K + t * _L, _L)
                    v0 = i0_v[b, sl]
                    v1 = i1_v[b, sl]
                    lin = v0 * 8 + v1
                    word = plsc.load_gather(
                        map_v, [lax.shift_right_logical(lin, 1)])
                    hi = lax.shift_right_logical(word, 16)
                    lo = jnp.bitwise_and(word, 0xFFFF)
                    m = jnp.where(jnp.bitwise_and(lin, 1) == 1, hi, lo)
                    m = jnp.minimum(m, num_rows - 4)
                    spec = (num_rows - 3) + jnp.minimum(jnp.maximum(v0, 0), 2)
                    row_v[b, j, pl.ds(t * _L, _L)] = jnp.where(
                        v0 <= 2, spec, m)
            @pl.when(i >= 1)
            def _():
                pltpu.make_async_copy(
                    rows_v.at[b], out_hbm.at[pl.ds(0, _C)], sem_o[b]).wait()
            hs = [pltpu.async_copy(table_hbm.at[row_v.at[b, j]],
                                   rows_v.at[b, pl.ds(j * _K, _K)], sem_g)
                  for j in range(_NK)]
            for h in hs:
                h.wait()
            pltpu.async_copy(rows_v.at[b], out_hbm.at[pl.ds(off, _C)],
                             sem_o[b])

        def pair_body(i, carry):
            process(i, 2 * i, 0)
            process(i, 2 * i + 1, 1)
            return carry

        lax.fori_loop(0, n_pairs, pair_body, 0)
        for b in (0, 1):
            pltpu.make_async_copy(
                rows_v.at[b], out_hbm.at[pl.ds(0, _C)], sem_o[b]).wait()
            pltpu.make_async_copy(
                i0_hbm.at[pl.ds(0, _C)], i0_v.at[b], sem_idx).wait()

    return lookup(i0, i1, map_packed, table)


def kernel(idx, special_table, attached_table, index_map):
    original_shape = idx.shape[:-1]
    n = 1
    for s in original_shape:
        n *= s
    flat = idx.reshape(n, 2)
    i0 = flat[:, 0].astype(jnp.int32)
    i1 = flat[:, 1].astype(jnp.int32)
    a = attached_table.shape[0]
    table = jnp.concatenate(
        [attached_table.astype(jnp.float32), special_table.astype(jnp.float32)],
        axis=0)
    mp = index_map.reshape(-1).astype(jnp.uint32)
    map_packed = (mp[0::2] | (mp[1::2] << 16)).astype(jnp.int32)
    out = _lookup_call(n, a + 3, map_packed.shape[0], i0, i1, map_packed,
                       table)
    return out.reshape(*original_shape, _D)


# speedup vs baseline: 31.3147x; 1.2786x over previous
import functools

import jax
import jax.numpy as jnp
from jax import lax
from jax.experimental import pallas as pl
from jax.experimental.pallas import tpu as pltpu
from jax.experimental.pallas import tpu_sc as plsc

_D = 64
_BT = 128
_L = 16
_WBS = _BT + 1


@functools.partial(jax.jit, static_argnums=(0, 1, 2))
def _lookup_call(hist, nbt, num_rows, j4, map_packed, table):
    info = plsc.get_sparse_core_info()
    nw = info.num_cores * info.num_subcores
    n_st = hist * nbt
    assert n_st % (2 * nw) == 0, (hist, nbt, nw)
    st_per_w = n_st // nw
    map_words = map_packed.shape[0]
    mesh = plsc.VectorSubcoreMesh(core_axis_name="c", subcore_axis_name="s")

    @functools.partial(
        pl.kernel,
        mesh=mesh,
        compiler_params=pltpu.CompilerParams(use_tc_tiling_on_sc=False,
                                             needs_layout_passes=False),
        out_type=jax.ShapeDtypeStruct((hist, _D // 8, nbt, 8, _BT),
                                      jnp.float32),
        scratch_types=[
            pltpu.VMEM((map_words,), jnp.int32),
            pltpu.VMEM((2, 2, _BT), jnp.int32),
            pltpu.VMEM((2, _BT), jnp.int32),
            pltpu.VMEM((2, _BT, _D), jnp.float32),
            pltpu.VMEM((_D, _WBS), jnp.float32),
            pltpu.SemaphoreType.DMA,
            pltpu.SemaphoreType.DMA,
        ],
    )
    def lookup(j4_hbm, map_hbm, table_hbm, out_hbm,
               map_v, idx_v, row_v, rows_v, wb_v, sem_idx, sem_g):
        wid = lax.axis_index("s") * info.num_cores + lax.axis_index("c")
        s0 = wid * st_per_w
        iota = lax.iota(jnp.int32, _L)

        pltpu.async_copy(j4_hbm.at[s0 // nbt, s0 % nbt], idx_v.at[0], sem_idx)
        pltpu.sync_copy(map_hbm, map_v)

        def compute_rowids(k, b):
            pltpu.make_async_copy(
                j4_hbm.at[0, 0], idx_v.at[b], sem_idx).wait()
            kn = jnp.minimum(k + 1, st_per_w - 1)
            sn = s0 + kn
            pltpu.async_copy(j4_hbm.at[sn // nbt, sn % nbt], idx_v.at[1 - b],
                             sem_idx)
            for t in range(_BT // _L):
                sl = pl.ds(t * _L, _L)
                v0 = idx_v[b, 0, sl]
                v1 = idx_v[b, 1, sl]
                lin = v0 * 8 + v1
                word = plsc.load_gather(
                    map_v, [lax.shift_right_logical(lin, 1)])
                hi = lax.shift_right_logical(word, 16)
                lo = jnp.bitwise_and(word, 0xFFFF)
                m = jnp.where(jnp.bitwise_and(lin, 1) == 1, hi, lo)
                m = jnp.minimum(m, num_rows - 4)
                spec = (num_rows - 3) + jnp.minimum(jnp.maximum(v0, 0), 2)
                row_v[b, sl] = jnp.where(v0 <= 2, spec, m)

        def fire_gather(b):
            pltpu.async_copy(table_hbm.at[row_v.at[b]], rows_v.at[b], sem_g)

        def drain_gather(b):
            pltpu.make_async_copy(
                table_hbm.at[pl.ds(0, _BT)], rows_v.at[b], sem_g).wait()

        def transpose_and_write(k, b):
            def tbody(bl, carry):
                for d0 in range(0, _D, _L):
                    v = rows_v[b, bl, pl.ds(d0, _L)]
                    plsc.store_scatter(wb_v, [d0 + iota, iota * 0 + bl], v)
                return carry
            lax.fori_loop(0, _BT, tbody, 0)
            s = s0 + k
            h = s // nbt
            bt = s % nbt
            for dt in range(_D // 8):
                pltpu.sync_copy(wb_v.at[pl.ds(dt * 8, 8), pl.ds(0, _BT)],
                                out_hbm.at[h, dt, bt])

        def stage(i, k, b, first):
            compute_rowids(k, b)
            fire_gather(b)
            if first:
                @pl.when(i >= 1)
                def _():
                    transpose_and_write(k - 1, 1 - b)
            else:
                transpose_and_write(k - 1, 1 - b)
            drain_gather(b)

        def pair_body(i, carry):
            stage(i, 2 * i, 0, True)
            stage(i, 2 * i + 1, 1, False)
            return carry

        lax.fori_loop(0, st_per_w // 2, pair_body, 0)
        transpose_and_write(st_per_w - 1, 1)
        pltpu.make_async_copy(j4_hbm.at[0, 0], idx_v.at[0], sem_idx).wait()

    return lookup(j4, map_packed, table)


def kernel(idx, special_table, attached_table, index_map):
    b, hist, _ = idx.shape
    nbt = b // _BT
    j4 = (idx.astype(jnp.int32)
          .transpose(1, 2, 0)
          .reshape(hist, 2, nbt, _BT)
          .transpose(0, 2, 1, 3))
    a = attached_table.shape[0]
    table = jnp.concatenate(
        [attached_table.astype(jnp.float32), special_table.astype(jnp.float32)],
        axis=0)
    mp = index_map.reshape(-1).astype(jnp.uint32)
    map_packed = (mp[0::2] | (mp[1::2] << 16)).astype(jnp.int32)
    u = _lookup_call(hist, nbt, a + 3, j4, map_packed, table)
    return u.transpose(2, 4, 0, 1, 3).reshape(b, hist, _D)


# speedup vs baseline: 37.7732x; 1.2062x over previous
import functools

import jax
import jax.numpy as jnp
from jax import lax
from jax.experimental import pallas as pl
from jax.experimental.pallas import tpu as pltpu
from jax.experimental.pallas import tpu_sc as plsc

_D = 64
_BT = 128
_L = 16
_WBS = _BT + 1


@functools.partial(jax.jit, static_argnums=(0, 1, 2))
def _lookup_call(hist, nbt, num_rows, j4, map_packed, table):
    info = plsc.get_sparse_core_info()
    nw = info.num_cores * info.num_subcores
    n_st = hist * nbt
    assert n_st % (2 * nw) == 0, (hist, nbt, nw)
    st_per_w = n_st // nw
    map_words = map_packed.shape[0]
    mesh = plsc.VectorSubcoreMesh(core_axis_name="c", subcore_axis_name="s")

    @functools.partial(
        pl.kernel,
        mesh=mesh,
        compiler_params=pltpu.CompilerParams(use_tc_tiling_on_sc=False,
                                             needs_layout_passes=False),
        out_type=jax.ShapeDtypeStruct((hist, _D // 8, nbt, 8, _BT),
                                      jnp.float32),
        scratch_types=[
            pltpu.VMEM((map_words,), jnp.int32),
            pltpu.VMEM((2, 2, _BT), jnp.int32),
            pltpu.VMEM((2, _BT), jnp.int32),
            pltpu.VMEM((2, _BT, _D), jnp.float32),
            pltpu.VMEM((2, _D, _WBS), jnp.float32),
            pltpu.SemaphoreType.DMA,
            pltpu.SemaphoreType.DMA,
            pltpu.SemaphoreType.DMA,
            pltpu.SemaphoreType.DMA,
        ],
    )
    def lookup(j4_hbm, map_hbm, table_hbm, out_hbm,
               map_v, idx_v, row_v, rows_v, wb_v, sem_idx, sem_g, sem_o0,
               sem_o1):
        wid = lax.axis_index("s") * info.num_cores + lax.axis_index("c")
        s0 = wid * st_per_w
        iota = lax.iota(jnp.int32, _L)
        drows = [iota + d0 for d0 in range(0, _D, _L)]
        sem_o = (sem_o0, sem_o1)

        pltpu.async_copy(j4_hbm.at[s0 // nbt, s0 % nbt], idx_v.at[0], sem_idx)
        pltpu.sync_copy(map_hbm, map_v)

        def compute_rowids(k, b):
            pltpu.make_async_copy(
                j4_hbm.at[0, 0], idx_v.at[b], sem_idx).wait()
            sn = s0 + jnp.minimum(k + 1, st_per_w - 1)
            pltpu.async_copy(j4_hbm.at[sn // nbt, sn % nbt], idx_v.at[1 - b],
                             sem_idx)
            for t in range(_BT // _L):
                sl = pl.ds(t * _L, _L)
                v0 = idx_v[b, 0, sl]
                v1 = idx_v[b, 1, sl]
                lin = v0 * 8 + v1
                word = plsc.load_gather(
                    map_v, [lax.shift_right_logical(lin, 1)])
                hi = lax.shift_right_logical(word, 16)
                lo = jnp.bitwise_and(word, 0xFFFF)
                m = jnp.where(jnp.bitwise_and(lin, 1) == 1, hi, lo)
                m = jnp.minimum(m, num_rows - 4)
                spec = (num_rows - 3) + jnp.minimum(jnp.maximum(v0, 0), 2)
                row_v[b, sl] = jnp.where(v0 <= 2, spec, m)

        def fire_gather(b):
            pltpu.async_copy(table_hbm.at[row_v.at[b]], rows_v.at[b], sem_g)

        def drain_gather(b):
            pltpu.make_async_copy(
                table_hbm.at[pl.ds(0, _BT)], rows_v.at[b], sem_g).wait()

        def drain_writeout(w):
            for dt in range(_D // 8):
                pltpu.make_async_copy(
                    wb_v.at[w, pl.ds(dt * 8, 8), pl.ds(0, _BT)],
                    out_hbm.at[0, dt, 0], sem_o[w]).wait()

        def transpose_and_write(k, b):
            def tbody(blk, carry):
                for j in range(_L):
                    bl = blk * _L + j
                    col = iota * 0 + bl
                    for i, d0 in enumerate(range(0, _D, _L)):
                        v = rows_v[b, bl, pl.ds(d0, _L)]
                        plsc.store_scatter(wb_v.at[b], [drows[i], col], v)
                return carry
            lax.fori_loop(0, _BT // _L, tbody, 0)
            s = s0 + k
            h = s // nbt
            bt = s % nbt
            for dt in range(_D // 8):
                pltpu.async_copy(
                    wb_v.at[b, pl.ds(dt * 8, 8), pl.ds(0, _BT)],
                    out_hbm.at[h, dt, bt], sem_o[b])

        def stage(i, k, b, min_i1, min_i2):
            compute_rowids(k, b)
            fire_gather(b)

            @pl.when(i >= min_i1)
            def _():
                @pl.when(i >= min_i2)
                def _():
                    drain_writeout(1 - b)
                transpose_and_write(k - 1, 1 - b)
            drain_gather(b)

        def pair_body(i, carry):
            stage(i, 2 * i, 0, 1, 2)
            stage(i, 2 * i + 1, 1, 0, 1)
            return carry

        lax.fori_loop(0, st_per_w // 2, pair_body, 0)
        drain_writeout(1)
        transpose_and_write(st_per_w - 1, 1)
        drain_writeout(0)
        drain_writeout(1)
        pltpu.make_async_copy(j4_hbm.at[0, 0], idx_v.at[0], sem_idx).wait()

    return lookup(j4, map_packed, table)


def kernel(idx, special_table, attached_table, index_map):
    b, hist, _ = idx.shape
    nbt = b // _BT
    j4 = (idx.astype(jnp.int32)
          .transpose(1, 2, 0)
          .reshape(hist, 2, nbt, _BT)
          .transpose(0, 2, 1, 3))
    a = attached_table.shape[0]
    table = jnp.concatenate(
        [attached_table.astype(jnp.float32), special_table.astype(jnp.float32)],
        axis=0)
    mp = index_map.reshape(-1).astype(jnp.uint32)
    map_packed = (mp[0::2] | (mp[1::2] << 16)).astype(jnp.int32)
    u = _lookup_call(hist, nbt, a + 3, j4, map_packed, table)
    return u.transpose(2, 4, 0, 1, 3).reshape(b, hist, _D)
